# Initial kernel scaffold; baseline (speedup 1.0000x reference)
import jax
import jax.numpy as jnp
from jax.experimental import pallas as pl


def kernel(texts_indices, table):
    raise NotImplementedError("write your pallas kernel here")


# speedup vs baseline: 5.6036x; 5.6036x over previous
import functools

import jax
import jax.numpy as jnp
from jax import lax
from jax.experimental import pallas as pl
from jax.experimental.pallas import tpu as pltpu
from jax.experimental.pallas import tpu_sc as plsc

VOCAB = 1000
D = 128
BATCH = 4096
SEQ = 200
B_TOTAL = BATCH * SEQ
NC, NS = 2, 16
NW = NC * NS
CHUNK = 128
ROWS_PER_W = B_TOTAL // (NW * CHUNK)
BASE_PER_W = B_TOTAL // NW


@functools.partial(
    pl.kernel,
    out_type=jax.ShapeDtypeStruct((B_TOTAL, D), jnp.float32),
    mesh=plsc.VectorSubcoreMesh(core_axis_name="c", subcore_axis_name="s"),
    scratch_types=[
        pltpu.VMEM((ROWS_PER_W, CHUNK), jnp.int32),
        pltpu.VMEM((CHUNK, D), jnp.float32),
        pltpu.SemaphoreType.DMA,
    ],
)
def _emb_lookup(idx_hbm, table_hbm, out_hbm, idx_v, rows_v, sem):
    wid = lax.axis_index("s") * NC + lax.axis_index("c")
    base = wid * BASE_PER_W
    pltpu.sync_copy(idx_hbm.at[pl.ds(wid * ROWS_PER_W, ROWS_PER_W)], idx_v)

    def step(j, carry):
        pltpu.async_copy(table_hbm.at[idx_v.at[j]], rows_v, sem).wait()
        pltpu.sync_copy(rows_v, out_hbm.at[pl.ds(base + j * CHUNK, CHUNK)])
        return carry

    lax.fori_loop(0, ROWS_PER_W, step, 0)


def kernel(texts_indices, table):
    idx = texts_indices.reshape(B_TOTAL // CHUNK, CHUNK).astype(jnp.int32)
    out = _emb_lookup(idx, table)
    return out.reshape(BATCH, SEQ, D)


# speedup vs baseline: 6.8521x; 1.2228x over previous
import functools

import jax
import jax.numpy as jnp
from jax import lax
from jax.experimental import pallas as pl
from jax.experimental.pallas import tpu as pltpu
from jax.experimental.pallas import tpu_sc as plsc

VOCAB = 1000
D = 128
BATCH = 4096
SEQ = 200
B_TOTAL = BATCH * SEQ
NC, NS = 2, 16
NW = NC * NS
CHUNK = 128
ROWS_PER_W = B_TOTAL // (NW * CHUNK)
BASE_PER_W = B_TOTAL // NW
NBUF = 4
GA = 2


@functools.partial(
    pl.kernel,
    out_type=jax.ShapeDtypeStruct((B_TOTAL, D), jnp.float32),
    mesh=plsc.VectorSubcoreMesh(core_axis_name="c", subcore_axis_name="s"),
    scratch_types=[
        pltpu.VMEM((ROWS_PER_W, CHUNK), jnp.int32),
        pltpu.VMEM((CHUNK, D), jnp.float32),
        pltpu.VMEM((CHUNK, D), jnp.float32),
        pltpu.VMEM((CHUNK, D), jnp.float32),
        pltpu.VMEM((CHUNK, D), jnp.float32),
        pltpu.SemaphoreType.DMA,
        pltpu.SemaphoreType.DMA,
        pltpu.SemaphoreType.DMA,
        pltpu.SemaphoreType.DMA,
        pltpu.SemaphoreType.DMA,
        pltpu.SemaphoreType.DMA,
        pltpu.SemaphoreType.DMA,
        pltpu.SemaphoreType.DMA,
    ],
)
def _emb_lookup(idx_hbm, table_hbm, out_hbm, idx_v,
                r0, r1, r2, r3, g0, g1, g2, g3, w0, w1, w2, w3):
    rows = (r0, r1, r2, r3)
    gsem = (g0, g1, g2, g3)
    wsem = (w0, w1, w2, w3)
    wid = lax.axis_index("s") * NC + lax.axis_index("c")
    base = wid * BASE_PER_W
    pltpu.sync_copy(idx_hbm.at[pl.ds(wid * ROWS_PER_W, ROWS_PER_W)], idx_v)

    def start_gather(j, b):
        pltpu.make_async_copy(table_hbm.at[idx_v.at[j]], rows[b], gsem[b]).start()

    def wait_gather(j, b):
        pltpu.make_async_copy(table_hbm.at[idx_v.at[j]], rows[b], gsem[b]).wait()

    def start_write(j, b):
        pltpu.make_async_copy(
            rows[b], out_hbm.at[pl.ds(base + j * CHUNK, CHUNK)], wsem[b]).start()

    def wait_write(b):
        pltpu.make_async_copy(
            rows[b], out_hbm.at[pl.ds(base, CHUNK)], wsem[b]).wait()

    def slot(j, b, first):
        wait_gather(j, b)
        start_write(j, b)
        nb = (b + GA) % NBUF
        if not first:
            wait_write(nb)
        start_gather(lax.rem(j + GA, ROWS_PER_W), nb)

    for b in range(GA):
        start_gather(b, b)
    for b in range(NBUF):
        slot(b, b, first=b < GA)

    def outer(i, carry):
        for b in range(NBUF):
            slot(i * NBUF + b, b, first=False)
        return carry

    lax.fori_loop(1, ROWS_PER_W // NBUF, outer, 0)

    for b in range(GA):
        wait_gather(b, b)
    for b in range(GA, NBUF):
        wait_write(b)


def kernel(texts_indices, table):
    idx = texts_indices.reshape(B_TOTAL // CHUNK, CHUNK).astype(jnp.int32)
    out = _emb_lookup(idx, table)
    return out.reshape(BATCH, SEQ, D)
